# baseline (speedup 1.0000x reference)
import functools

import jax
import jax.numpy as jnp
from jax import lax
from jax.experimental import pallas as pl
from jax.experimental.pallas import tpu as pltpu
from jax.experimental.pallas import tpu_sc as plsc

D = 64
NC, NS = 2, 16
NW = NC * NS
CH = 128
S = 50
NB = 5


@jax.jit
def _gather_rows(idx, table):
    mesh = plsc.VectorSubcoreMesh(core_axis_name="c", subcore_axis_name="s")

    @functools.partial(
        pl.kernel,
        out_type=jax.ShapeDtypeStruct((S, 8, NW, 8, CH), jnp.float32),
        mesh=mesh,
        scratch_types=[
            pltpu.VMEM((S, CH), jnp.int32),
            pltpu.VMEM((NB, CH, D), jnp.float32),
            pltpu.VMEM((NB, D, 129), jnp.float32),
            pltpu.VMEM((CH, 16), jnp.int32),
        ] + [pltpu.SemaphoreType.DMA] * (2 * NB),
        compiler_params=pltpu.CompilerParams(
            use_tc_tiling_on_sc=False, needs_layout_passes=False
        ),
    )
    def k(idx_hbm, table_hbm, out_hbm, idx_v, gbuf, tbuf, cvt, *sems):
        wid = lax.axis_index("s") * NC + lax.axis_index("c")
        pltpu.sync_copy(idx_hbm.at[:, wid], idx_v)
        iota = lax.iota(jnp.int32, 16)
        gsems, wsems = sems[:NB], sems[NB:]

        def gfire(sq, p):
            pltpu.async_copy(table_hbm.at[idx_v.at[sq]], gbuf.at[p], gsems[p])

        def gwait(sq, p):
            pltpu.make_async_copy(
                table_hbm.at[idx_v.at[sq]], gbuf.at[p], gsems[p]
            ).wait()

        def wfire(sq, p):
            for tr in range(8):
                pltpu.async_copy(
                    tbuf.at[p, pl.ds(8 * tr, 8), pl.ds(0, CH)],
                    out_hbm.at[sq, tr, wid],
                    wsems[p],
                )

        def wwait(sq, p):
            for tr in range(8):
                pltpu.make_async_copy(
                    tbuf.at[p, pl.ds(8 * tr, 8), pl.ds(0, CH)],
                    out_hbm.at[sq, tr, wid],
                    wsems[p],
                ).wait()

        djs = [iota + 16 * j for j in range(D // 16)]

        for c in range(CH):
            cvt[c] = jnp.full((16,), c, jnp.int32)

        def transpose(p):
            for c in range(CH):
                cv = cvt[c]
                for j in range(D // 16):
                    v = gbuf[p, c, pl.ds(16 * j, 16)]
                    plsc.store_scatter(tbuf.at[p], [djs[j], cv], v)

        for q in range(NB - 1):
            gfire(q, q)

        def body(i, carry):
            s0 = NB * i
            for q in range(NB):
                s = s0 + q
                gwait(s, q)

                @pl.when(s + NB - 1 < S)
                def _g():
                    gfire(s + NB - 1, (q + NB - 1) % NB)

                @pl.when(i > 0)
                def _w():
                    wwait(s - NB, q)

                transpose(q)
                wfire(s, q)
            return carry

        lax.fori_loop(0, S // NB, body, 0, unroll=False)
        for q in range(NB):
            wwait(S - NB + q, q)

    return k(idx, table)


def kernel(x, emb_weight):
    idx = x.T.reshape(S, NW, CH)
    out5 = _gather_rows(idx, emb_weight)
    out = out5.transpose(0, 1, 3, 2, 4).reshape(S, D, NW * CH).transpose(2, 0, 1)
    return out


# speedup vs baseline: 1.0605x; 1.0605x over previous
import functools

import jax
import jax.numpy as jnp
from jax import lax
from jax.experimental import pallas as pl
from jax.experimental.pallas import tpu as pltpu
from jax.experimental.pallas import tpu_sc as plsc

D = 64
NC, NS = 2, 16
NW = NC * NS
CH = 128
S = 50
NB = 2


@jax.jit
def _gather_rows(idx, table):
    mesh = plsc.VectorSubcoreMesh(core_axis_name="c", subcore_axis_name="s")

    @functools.partial(
        pl.kernel,
        out_type=jax.ShapeDtypeStruct((S, 8, NW, 8, CH), jnp.float32),
        mesh=mesh,
        scratch_types=[
            pltpu.VMEM((S, CH), jnp.int32),
            pltpu.VMEM((NB, CH, D), jnp.float32),
            pltpu.VMEM((NB, D, 129), jnp.float32),
            pltpu.VMEM((CH, 16), jnp.int32),
        ] + [pltpu.SemaphoreType.DMA] * (2 * NB),
        compiler_params=pltpu.CompilerParams(
            use_tc_tiling_on_sc=False, needs_layout_passes=False
        ),
    )
    def k(idx_hbm, table_hbm, out_hbm, idx_v, gbuf, tbuf, cvt, *sems):
        wid = lax.axis_index("s") * NC + lax.axis_index("c")
        pltpu.sync_copy(idx_hbm.at[:, wid], idx_v)
        iota = lax.iota(jnp.int32, 16)
        gsems, wsems = sems[:NB], sems[NB:]

        def gfire(sq, p):
            pltpu.async_copy(table_hbm.at[idx_v.at[sq]], gbuf.at[p], gsems[p])

        def gwait(sq, p):
            pltpu.make_async_copy(
                table_hbm.at[idx_v.at[sq]], gbuf.at[p], gsems[p]
            ).wait()

        def wfire(sq, p):
            for tr in range(8):
                pltpu.async_copy(
                    tbuf.at[p, pl.ds(8 * tr, 8), pl.ds(0, CH)],
                    out_hbm.at[sq, tr, wid],
                    wsems[p],
                )

        def wwait(sq, p):
            for tr in range(8):
                pltpu.make_async_copy(
                    tbuf.at[p, pl.ds(8 * tr, 8), pl.ds(0, CH)],
                    out_hbm.at[sq, tr, wid],
                    wsems[p],
                ).wait()

        djs = [iota + 16 * j for j in range(D // 16)]

        for c in range(CH):
            cvt[c] = jnp.full((16,), c, jnp.int32)

        def transpose(p):
            for c in range(CH):
                cv = cvt[c]
                for j in range(D // 16):
                    v = gbuf[p, c, pl.ds(16 * j, 16)]
                    plsc.store_scatter(tbuf.at[p], [djs[j], cv], v)

        for q in range(NB - 1):
            gfire(q, q)

        def body(i, carry):
            s0 = NB * i
            for q in range(NB):
                s = s0 + q
                gwait(s, q)

                @pl.when(s + NB - 1 < S)
                def _g():
                    gfire(s + NB - 1, (q + NB - 1) % NB)

                @pl.when(i > 0)
                def _w():
                    wwait(s - NB, q)

                transpose(q)
                wfire(s, q)
            return carry

        lax.fori_loop(0, S // NB, body, 0, unroll=False)
        for q in range(NB):
            wwait(S - NB + q, q)

    return k(idx, table)


def kernel(x, emb_weight):
    idx = x.T.reshape(S, NW, CH)
    out5 = _gather_rows(idx, emb_weight)
    out = out5.transpose(0, 1, 3, 2, 4).reshape(S, D, NW * CH).transpose(2, 0, 1)
    return out


# speedup vs baseline: 1.2429x; 1.1720x over previous
import functools

import jax
import jax.numpy as jnp
from jax import lax
from jax.experimental import pallas as pl
from jax.experimental.pallas import tpu as pltpu
from jax.experimental.pallas import tpu_sc as plsc

D = 64
NC, NS = 2, 16
NW = NC * NS
CH = 128
S = 50
NB = 2


@jax.jit
def _gather_rows(idx, table):
    mesh = plsc.VectorSubcoreMesh(core_axis_name="c", subcore_axis_name="s")

    @functools.partial(
        pl.kernel,
        out_type=jax.ShapeDtypeStruct((S, 8, NW, 8, CH), jnp.float32),
        mesh=mesh,
        scratch_types=[
            pltpu.VMEM((S, CH), jnp.int32),
            pltpu.VMEM((NB, CH, D), jnp.float32),
            pltpu.VMEM((NB, D, 129), jnp.float32),
            pltpu.VMEM((CH, 16), jnp.int32),
        ] + [pltpu.SemaphoreType.DMA] * (2 * NB),
        compiler_params=pltpu.CompilerParams(
            use_tc_tiling_on_sc=False, needs_layout_passes=False
        ),
    )
    def k(idx_hbm, table_hbm, out_hbm, idx_v, gbuf, tbuf, cvt, *sems):
        wid = lax.axis_index("s") * NC + lax.axis_index("c")
        pltpu.sync_copy(idx_hbm.at[:, wid], idx_v)
        iota = lax.iota(jnp.int32, 16)
        gsems, wsems = sems[:NB], sems[NB:]

        def gfire(sq, p):
            pltpu.async_copy(table_hbm.at[idx_v.at[sq]], gbuf.at[p], gsems[p])

        def gwait(sq, p):
            pltpu.make_async_copy(
                table_hbm.at[idx_v.at[sq]], gbuf.at[p], gsems[p]
            ).wait()

        def wfire(sq, p):
            for tr in range(8):
                pltpu.async_copy(
                    tbuf.at[p, pl.ds(8 * tr, 8), pl.ds(0, CH)],
                    out_hbm.at[sq, tr, wid],
                    wsems[p],
                )

        def wwait(sq, p):
            for tr in range(8):
                pltpu.make_async_copy(
                    tbuf.at[p, pl.ds(8 * tr, 8), pl.ds(0, CH)],
                    out_hbm.at[sq, tr, wid],
                    wsems[p],
                ).wait()

        djs = [iota + 16 * j for j in range(D // 16)]

        for c in range(CH):
            cvt[c] = jnp.full((16,), c, jnp.int32)

        nj = D // 16

        def transpose(p):
            cv = cvt[0]
            vs = [gbuf[p, 0, pl.ds(16 * j, 16)] for j in range(nj)]
            for c in range(CH):
                cv_c, vs_c = cv, vs
                if c + 1 < CH:
                    cv = cvt[c + 1]
                    vs = [gbuf[p, c + 1, pl.ds(16 * j, 16)] for j in range(nj)]
                for j in range(nj):
                    plsc.store_scatter(tbuf.at[p], [djs[j], cv_c], vs_c[j])

        for q in range(NB - 1):
            gfire(q, q)

        def body(i, carry):
            s0 = NB * i
            for q in range(NB):
                s = s0 + q
                gwait(s, q)

                @pl.when(s + NB - 1 < S)
                def _g():
                    gfire(s + NB - 1, (q + NB - 1) % NB)

                @pl.when(i > 0)
                def _w():
                    wwait(s - NB, q)

                transpose(q)
                wfire(s, q)
            return carry

        lax.fori_loop(0, S // NB, body, 0, unroll=False)
        for q in range(NB):
            wwait(S - NB + q, q)

    return k(idx, table)


def kernel(x, emb_weight):
    idx = x.T.reshape(S, NW, CH)
    out5 = _gather_rows(idx, emb_weight)
    out = out5.transpose(0, 1, 3, 2, 4).reshape(S, D, NW * CH).transpose(2, 0, 1)
    return out


# speedup vs baseline: 1.2696x; 1.0215x over previous
import functools

import jax
import jax.numpy as jnp
from jax import lax
from jax.experimental import pallas as pl
from jax.experimental.pallas import tpu as pltpu
from jax.experimental.pallas import tpu_sc as plsc

D = 64
NC, NS = 2, 16
NW = NC * NS
CH = 128
S = 50
NB = 2


@jax.jit
def _gather_rows(idx, table):
    mesh = plsc.VectorSubcoreMesh(core_axis_name="c", subcore_axis_name="s")

    @functools.partial(
        pl.kernel,
        out_type=jax.ShapeDtypeStruct((S, 8, NW, 8, CH), jnp.float32),
        mesh=mesh,
        scratch_types=[
            pltpu.VMEM((S, CH), jnp.int32),
            pltpu.VMEM((NB, CH, D), jnp.float32),
            pltpu.VMEM((NB, D, 129), jnp.float32),
            pltpu.VMEM((CH, 16), jnp.int32),
        ] + [pltpu.SemaphoreType.DMA] * (2 * NB),
        compiler_params=pltpu.CompilerParams(
            use_tc_tiling_on_sc=False, needs_layout_passes=False
        ),
    )
    def k(idx_hbm, table_hbm, out_hbm, idx_v, gbuf, tbuf, cvt, *sems):
        wid = lax.axis_index("s") * NC + lax.axis_index("c")
        pltpu.sync_copy(idx_hbm.at[:, wid], idx_v)
        iota = lax.iota(jnp.int32, 16)
        gsems, wsems = sems[:NB], sems[NB:]

        def gfire(sq, p):
            pltpu.async_copy(table_hbm.at[idx_v.at[sq]], gbuf.at[p], gsems[p])

        def gwait(sq, p):
            pltpu.make_async_copy(
                table_hbm.at[idx_v.at[sq]], gbuf.at[p], gsems[p]
            ).wait()

        def wfire(sq, p):
            for tr in range(8):
                pltpu.async_copy(
                    tbuf.at[p, pl.ds(8 * tr, 8), pl.ds(0, CH)],
                    out_hbm.at[sq, tr, wid],
                    wsems[p],
                )

        def wwait(sq, p):
            for tr in range(8):
                pltpu.make_async_copy(
                    tbuf.at[p, pl.ds(8 * tr, 8), pl.ds(0, CH)],
                    out_hbm.at[sq, tr, wid],
                    wsems[p],
                ).wait()

        djs = [iota + 16 * j for j in range(D // 16)]
        ones = jnp.full((16,), 1, jnp.int32)

        for c in range(CH):
            cvt[c] = jnp.full((16,), c, jnp.int32)

        nj = D // 16

        def transpose(p):
            cv = cvt[0]
            vs = [gbuf[p, 0, pl.ds(16 * j, 16)] for j in range(nj)]
            for c in range(CH):
                cv_c, vs_c = cv, vs
                if c + 1 < CH:
                    cv = cvt[c + 1] if (c + 1) % 16 == 0 else cv_c + ones
                    vs = [gbuf[p, c + 1, pl.ds(16 * j, 16)] for j in range(nj)]
                for j in range(nj):
                    plsc.store_scatter(tbuf.at[p], [djs[j], cv_c], vs_c[j])

        for q in range(NB - 1):
            gfire(q, q)

        def body(i, carry):
            s0 = NB * i
            for q in range(NB):
                s = s0 + q
                gwait(s, q)

                @pl.when(s + NB - 1 < S)
                def _g():
                    gfire(s + NB - 1, (q + NB - 1) % NB)

                @pl.when(i > 0)
                def _w():
                    wwait(s - NB, q)

                transpose(q)
                wfire(s, q)
            return carry

        lax.fori_loop(0, S // NB, body, 0, unroll=False)
        for q in range(NB):
            wwait(S - NB + q, q)

    return k(idx, table)


def kernel(x, emb_weight):
    idx = x.T.reshape(S, NW, CH)
    out5 = _gather_rows(idx, emb_weight)
    out = out5.transpose(0, 1, 3, 2, 4).reshape(S, D, NW * CH).transpose(2, 0, 1)
    return out


# speedup vs baseline: 1.3880x; 1.0932x over previous
import functools

import jax
import jax.numpy as jnp
from jax import lax
from jax.experimental import pallas as pl
from jax.experimental.pallas import tpu as pltpu
from jax.experimental.pallas import tpu_sc as plsc

D = 64
NC, NS = 2, 16
NW = NC * NS
CH = 128
S = 50
NB = 2


@jax.jit
def _gather_rows(idx, table):
    mesh = plsc.VectorSubcoreMesh(core_axis_name="c", subcore_axis_name="s")

    @functools.partial(
        pl.kernel,
        out_type=jax.ShapeDtypeStruct((S, 8, NW, 8, CH), jnp.float32),
        mesh=mesh,
        scratch_types=[
            pltpu.VMEM((S, CH), jnp.int32),
            pltpu.VMEM((NB, CH, D), jnp.float32),
            pltpu.VMEM((NB, D, 129), jnp.float32),
            pltpu.VMEM((CH, 16), jnp.int32),
        ] + [pltpu.SemaphoreType.DMA] * (2 * NB),
        compiler_params=pltpu.CompilerParams(
            use_tc_tiling_on_sc=False, needs_layout_passes=False
        ),
    )
    def k(idx_hbm, table_hbm, out_hbm, idx_v, gbuf, tbuf, cvt, *sems):
        wid = lax.axis_index("s") * NC + lax.axis_index("c")
        pltpu.sync_copy(idx_hbm.at[:, wid], idx_v)
        iota = lax.iota(jnp.int32, 16)
        gsems, wsems = sems[:NB], sems[NB:]

        def gfire(sq, p):
            pltpu.async_copy(table_hbm.at[idx_v.at[sq]], gbuf.at[p], gsems[p])

        def gwait(sq, p):
            pltpu.make_async_copy(
                table_hbm.at[idx_v.at[sq]], gbuf.at[p], gsems[p]
            ).wait()

        def wfire(sq, p):
            for tr in range(8):
                pltpu.async_copy(
                    tbuf.at[p, pl.ds(8 * tr, 8), pl.ds(0, CH)],
                    out_hbm.at[sq, tr, wid],
                    wsems[p],
                )

        def wwait(sq, p):
            for tr in range(8):
                pltpu.make_async_copy(
                    tbuf.at[p, pl.ds(8 * tr, 8), pl.ds(0, CH)],
                    out_hbm.at[sq, tr, wid],
                    wsems[p],
                ).wait()

        djs = [iota + 16 * j for j in range(D // 16)]
        ones = jnp.full((16,), 1, jnp.int32)

        for c in range(CH):
            cvt[c] = jnp.full((16,), c, jnp.int32)

        nj = D // 16

        def transpose(p):
            cv = cvt[0]
            vs = [gbuf[p, 0, pl.ds(16 * j, 16)] for j in range(nj)]
            for c in range(CH):
                cv_c, vs_c = cv, vs
                if c + 1 < CH:
                    cv = cvt[c + 1] if (c + 1) % 16 == 0 else cv_c + ones
                    vs = []
                for j in range(nj):
                    if c + 1 < CH:
                        vs.append(gbuf[p, c + 1, pl.ds(16 * j, 16)])
                    plsc.store_scatter(tbuf.at[p], [djs[j], cv_c], vs_c[j])

        for q in range(NB - 1):
            gfire(q, q)

        def body(i, carry):
            s0 = NB * i
            for q in range(NB):
                s = s0 + q
                gwait(s, q)

                @pl.when(s + NB - 1 < S)
                def _g():
                    gfire(s + NB - 1, (q + NB - 1) % NB)

                @pl.when(i > 0)
                def _w():
                    wwait(s - NB, q)

                transpose(q)
                wfire(s, q)
            return carry

        lax.fori_loop(0, S // NB, body, 0, unroll=False)
        for q in range(NB):
            wwait(S - NB + q, q)

    return k(idx, table)


def kernel(x, emb_weight):
    idx = x.T.reshape(S, NW, CH)
    out5 = _gather_rows(idx, emb_weight)
    out = out5.transpose(0, 1, 3, 2, 4).reshape(S, D, NW * CH).transpose(2, 0, 1)
    return out
